# baseline (speedup 1.0000x reference)
import functools

import jax
import jax.numpy as jnp
from jax import lax
from jax.experimental import pallas as pl
from jax.experimental.pallas import tpu as pltpu
from jax.experimental.pallas import tpu_sc as plsc

_N = 10000
_E = 320000
_H = 128
_ED = 16
_NC = 2
_NS = 16
_NW = _NC * _NS
_EPW = _E // _NW
_C = 40
_NCHUNK = _EPW // _C
_L = 16
_WT = 10
_RPT = _N // _WT
_ZR = 200
_SUP = 50
_NSUP = _NCHUNK // _SUP


def _proj2(x, wlT, wrT):
    m = x.shape[0]
    bm = 1000

    def body(x_ref, wl_ref, wr_ref, xl_ref, xr_ref):
        xb = x_ref[...]
        xl_ref[...] = jnp.dot(xb, wl_ref[...], preferred_element_type=jnp.float32)
        xr_ref[...] = jnp.dot(xb, wr_ref[...], preferred_element_type=jnp.float32)

    return pl.pallas_call(
        body,
        grid=(m // bm,),
        in_specs=[
            pl.BlockSpec((bm, x.shape[1]), lambda i: (i, 0)),
            pl.BlockSpec((x.shape[1], _H), lambda i: (0, 0)),
            pl.BlockSpec((x.shape[1], _H), lambda i: (0, 0)),
        ],
        out_specs=[
            pl.BlockSpec((bm, _H), lambda i: (i, 0)),
            pl.BlockSpec((bm, _H), lambda i: (i, 0)),
        ],
        out_shape=[jax.ShapeDtypeStruct((m, _H), jnp.float32)] * 2,
    )(x, wlT, wrT)


def _edge_el(edge_attr, weT):
    be = 2000

    def body(a_ref, w_ref, o_ref):
        o_ref[...] = jnp.dot(a_ref[...], w_ref[...], preferred_element_type=jnp.float32)

    return pl.pallas_call(
        body,
        grid=(_E // be,),
        in_specs=[
            pl.BlockSpec((be, _ED), lambda i: (i, 0)),
            pl.BlockSpec((_ED, _H), lambda i: (0, 0)),
        ],
        out_specs=pl.BlockSpec((be, _H), lambda i: (i, 0)),
        out_shape=jax.ShapeDtypeStruct((_E, _H), jnp.float32),
    )(edge_attr, weT)


def _combine_elu_stats(parts, dens, b):
    bm = 1000

    def body(p_ref, d_ref, b_ref, h_ref, s_ref):
        i = pl.program_id(0)
        raw = p_ref[0] + p_ref[1]
        den = d_ref[0, :, 0] + d_ref[1, :, 0] + 1e-16
        h = raw / den[:, None] + b_ref[...]
        h = jnp.where(h > 0, h, jnp.exp(h) - 1.0)
        h_ref[...] = h

        @pl.when(i == 0)
        def _():
            s_ref[...] = jnp.zeros_like(s_ref)

        s_ref[0:1, :] += jnp.sum(h, axis=0, keepdims=True)
        s_ref[1:2, :] += jnp.sum(h * h, axis=0, keepdims=True)

    return pl.pallas_call(
        body,
        grid=(_N // bm,),
        in_specs=[
            pl.BlockSpec((_NC, bm, _H), lambda i: (0, i, 0)),
            pl.BlockSpec((_NC, bm, _L), lambda i: (0, i, 0)),
            pl.BlockSpec((1, _H), lambda i: (0, 0)),
        ],
        out_specs=[
            pl.BlockSpec((bm, _H), lambda i: (i, 0)),
            pl.BlockSpec((2, _H), lambda i: (0, 0)),
        ],
        out_shape=[
            jax.ShapeDtypeStruct((_N, _H), jnp.float32),
            jax.ShapeDtypeStruct((2, _H), jnp.float32),
        ],
    )(parts, dens, b)


def _bn_proj2(h, stats, g, be, wlT, wrT):
    bm = 1000

    def body(h_ref, s_ref, g_ref, be_ref, wl_ref, wr_ref, xl_ref, xr_ref):
        mu = s_ref[0:1, :] * (1.0 / _N)
        var = s_ref[1:2, :] * (1.0 / _N) - mu * mu
        inv = g_ref[...] * lax.rsqrt(var + 1e-5)
        hb = (h_ref[...] - mu) * inv + be_ref[...]
        xl_ref[...] = jnp.dot(hb, wl_ref[...], preferred_element_type=jnp.float32)
        xr_ref[...] = jnp.dot(hb, wr_ref[...], preferred_element_type=jnp.float32)

    return pl.pallas_call(
        body,
        grid=(_N // bm,),
        in_specs=[
            pl.BlockSpec((bm, _H), lambda i: (i, 0)),
            pl.BlockSpec((2, _H), lambda i: (0, 0)),
            pl.BlockSpec((1, _H), lambda i: (0, 0)),
            pl.BlockSpec((1, _H), lambda i: (0, 0)),
            pl.BlockSpec((_H, _H), lambda i: (0, 0)),
            pl.BlockSpec((_H, _H), lambda i: (0, 0)),
        ],
        out_specs=[
            pl.BlockSpec((bm, _H), lambda i: (i, 0)),
            pl.BlockSpec((bm, _H), lambda i: (i, 0)),
        ],
        out_shape=[jax.ShapeDtypeStruct((_N, _H), jnp.float32)] * 2,
    )(h, stats, g, be, wlT, wrT)


def _final_combine(parts, dens, b):
    bm = 1000

    def body(p_ref, d_ref, b_ref, o_ref):
        raw = p_ref[0] + p_ref[1]
        den = d_ref[0, :, 0] + d_ref[1, :, 0] + 1e-16
        o_ref[...] = raw / den[:, None] + b_ref[...]

    return pl.pallas_call(
        body,
        grid=(_N // bm,),
        in_specs=[
            pl.BlockSpec((_NC, bm, _H), lambda i: (0, i, 0)),
            pl.BlockSpec((_NC, bm, _L), lambda i: (0, i, 0)),
            pl.BlockSpec((1, _H), lambda i: (0, 0)),
        ],
        out_specs=pl.BlockSpec((bm, _H), lambda i: (i, 0)),
        out_shape=jax.ShapeDtypeStruct((_N, _H), jnp.float32),
    )(parts, dens, b)


def _make_sc_layer(with_el):
    mesh = plsc.VectorSubcoreMesh(
        core_axis_name="c", subcore_axis_name="s",
        num_cores=_NC, num_subcores=_NS)

    scratch = [
        pltpu.VMEM((_SUP, _C), jnp.int32),
        pltpu.VMEM((_SUP, _C), jnp.int32),
        pltpu.VMEM((2, _C, _H), jnp.float32),
        pltpu.VMEM((2, _C, _H), jnp.float32),
        pltpu.VMEM((2, _C, _H), jnp.float32),
        pltpu.VMEM((2, _C, _L), jnp.float32),
        pltpu.VMEM((_H,), jnp.float32),
        pltpu.VMEM_SHARED((_N, _H), jnp.float32),
        pltpu.VMEM_SHARED((_N, _L), jnp.float32),
        pltpu.SemaphoreType.DMA,
        pltpu.SemaphoreType.DMA,
        pltpu.SemaphoreType.DMA,
        pltpu.SemaphoreType.DMA,
    ]
    out_type = (
        jax.ShapeDtypeStruct((_NC, _N, _H), jnp.float32),
        jax.ShapeDtypeStruct((_NC, _N, _L), jnp.float32),
    )

    def body(*refs):
        if with_el:
            (xl_hbm, xr_hbm, el_hbm, src_hbm, dst_hbm, att_hbm,
             zo_hbm, zd_hbm,
             out_hbm, den_hbm,
             srcw_v, dstw_v, xl_v, xr_v, el_v, p_v, att_v,
             out_sh, den_sh, sg0, sg1, ss0, ss1) = refs
        else:
            (xl_hbm, xr_hbm, src_hbm, dst_hbm, att_hbm,
             zo_hbm, zd_hbm,
             out_hbm, den_hbm,
             srcw_v, dstw_v, xl_v, xr_v, el_v, p_v, att_v,
             out_sh, den_sh, sg0, sg1, ss0, ss1) = refs

        cid = lax.axis_index("c")
        sid = lax.axis_index("s")
        wid = sid * _NC + cid
        sg = (sg0, sg1)
        ss = (ss0, ss1)

        pltpu.sync_copy(att_hbm, att_v)
        atts = [att_v[pl.ds(j * _L, _L)] for j in range(_H // _L)]

        @pl.when(sid < _WT)
        def _():
            for i in range(_RPT // _ZR):
                pltpu.sync_copy(zo_hbm, out_sh.at[pl.ds(sid * _RPT + i * _ZR, _ZR)])
                pltpu.sync_copy(zd_hbm, den_sh.at[pl.ds(sid * _RPT + i * _ZR, _ZR)])

        plsc.subcore_barrier()

        def issue_g(s, c, b):
            if with_el:
                off = wid * _EPW + s * (_SUP * _C) + c * _C
                pltpu.async_copy(el_hbm.at[pl.ds(off, _C)], el_v.at[b], sg[b])
            pltpu.async_copy(xl_hbm.at[srcw_v.at[c]], xl_v.at[b], sg[b])
            pltpu.async_copy(xr_hbm.at[dstw_v.at[c]], xr_v.at[b], sg[b])

        def wait_g(b):
            if with_el:
                pltpu.make_async_copy(el_hbm.at[pl.ds(0, _C)], el_v.at[b], sg[b]).wait()
            pltpu.make_async_copy(xl_hbm.at[srcw_v.at[0]], xl_v.at[b], sg[b]).wait()
            pltpu.make_async_copy(xr_hbm.at[dstw_v.at[0]], xr_v.at[b], sg[b]).wait()

        def issue_s(c, b):
            pltpu.async_copy(xr_v.at[b], out_sh.at[dstw_v.at[c]], ss[b], add=True)
            pltpu.async_copy(p_v.at[b], den_sh.at[dstw_v.at[c]], ss[b], add=True)

        def wait_s(b):
            pltpu.make_async_copy(xr_v.at[b], out_sh.at[dstw_v.at[0]], ss[b]).wait()
            pltpu.make_async_copy(p_v.at[b], den_sh.at[dstw_v.at[0]], ss[b]).wait()

        def compute(b):
            xlb, xrb, elb, pb = xl_v.at[b], xr_v.at[b], el_v.at[b], p_v.at[b]

            @plsc.parallel_loop(0, _C, unroll=8)
            def edge_a(i):
                acc = jnp.zeros((_L,), jnp.float32)
                xls = []
                for j in range(_H // _L):
                    sl = pl.ds(j * _L, _L)
                    xlv = xlb[i, sl]
                    xls.append(xlv)
                    mv = xlv + xrb[i, sl]
                    if with_el:
                        mv = mv + elb[i, sl]
                    mv = jnp.maximum(mv, mv * 0.2)
                    acc = acc + mv * atts[j]
                pvec = jnp.exp(jnp.broadcast_to(jnp.sum(acc), (_L,)))
                pb[i, :] = pvec
                for j in range(_H // _L):
                    xrb[i, pl.ds(j * _L, _L)] = xls[j] * pvec

        def superchunk(s, _):
            pltpu.sync_copy(src_hbm.at[wid, s], srcw_v)
            pltpu.sync_copy(dst_hbm.at[wid, s], dstw_v)
            issue_g(s, 0, 0)

            def pair(t, _):
                for b in (0, 1):
                    c = 2 * t + b
                    wait_g(b)

                    @pl.when(c < _SUP - 1)
                    def _():
                        @pl.when(c >= 1)
                        def _():
                            wait_s(1 - b)
                        issue_g(s, c + 1, 1 - b)

                    compute(b)
                    issue_s(c, b)
                return 0
            lax.fori_loop(0, _SUP // 2, pair, 0)
            wait_s(0)
            wait_s(1)
            return 0
        lax.fori_loop(0, _NSUP, superchunk, 0)

        plsc.subcore_barrier()

        @pl.when(sid < _WT)
        def _():
            pltpu.sync_copy(out_sh.at[pl.ds(sid * _RPT, _RPT)],
                            out_hbm.at[cid, pl.ds(sid * _RPT, _RPT)])
            pltpu.sync_copy(den_sh.at[pl.ds(sid * _RPT, _RPT)],
                            den_hbm.at[cid, pl.ds(sid * _RPT, _RPT)])

    return pl.kernel(
        body, out_type=out_type, mesh=mesh, scratch_types=scratch,
        compiler_params=pltpu.CompilerParams(
            needs_layout_passes=False, use_tc_tiling_on_sc=False))


_sc_layer_el = _make_sc_layer(True)
_sc_layer_plain = _make_sc_layer(False)


def kernel(x, edge_index, edge_attr, Wl1, Wr1, We1, att1, b1, g0, be0,
           Wl2, Wr2, att2, b2):
    src = edge_index[0].reshape(_NW, _NSUP, _SUP, _C)
    dst = edge_index[1].reshape(_NW, _NSUP, _SUP, _C)
    zo = jnp.zeros((_ZR, _H), jnp.float32)
    zd = jnp.zeros((_ZR, _L), jnp.float32)

    xl1, xr1 = _proj2(x, Wl1.T, Wr1.T)
    el1 = _edge_el(edge_attr, We1.T)
    parts1, dens1 = _sc_layer_el(xl1, xr1, el1, src, dst, att1, zo, zd)
    h, stats = _combine_elu_stats(parts1, dens1, b1.reshape(1, _H))
    xl2, xr2 = _bn_proj2(h, stats, g0.reshape(1, _H), be0.reshape(1, _H),
                         Wl2.T, Wr2.T)
    parts2, dens2 = _sc_layer_plain(xl2, xr2, src, dst, att2, zo, zd)
    return _final_combine(parts2, dens2, b2.reshape(1, _H))


# speedup vs baseline: 1.0837x; 1.0837x over previous
import functools

import jax
import jax.numpy as jnp
from jax import lax
from jax.experimental import pallas as pl
from jax.experimental.pallas import tpu as pltpu
from jax.experimental.pallas import tpu_sc as plsc

_N = 10000
_E = 320000
_H = 128
_ED = 16
_NC = 2
_NS = 16
_NW = _NC * _NS
_EPW = _E // _NW
_C = 40
_NCHUNK = _EPW // _C
_L = 16
_WT = 10
_RPT = _N // _WT
_ZR = 200
_SUP = 50
_NSUP = _NCHUNK // _SUP


def _proj2_el(x, wlT, wrT, edge_attr, weT):
    bm = 400
    be = _E // (_N // bm)

    def body(x_ref, wl_ref, wr_ref, a_ref, we_ref, xl_ref, xr_ref, el_ref):
        xb = x_ref[...]
        xl_ref[...] = jnp.dot(xb, wl_ref[...], preferred_element_type=jnp.float32)
        xr_ref[...] = jnp.dot(xb, wr_ref[...], preferred_element_type=jnp.float32)
        el_ref[...] = jnp.dot(a_ref[...], we_ref[...],
                              preferred_element_type=jnp.float32)

    return pl.pallas_call(
        body,
        grid=(_N // bm,),
        in_specs=[
            pl.BlockSpec((bm, x.shape[1]), lambda i: (i, 0)),
            pl.BlockSpec((x.shape[1], _H), lambda i: (0, 0)),
            pl.BlockSpec((x.shape[1], _H), lambda i: (0, 0)),
            pl.BlockSpec((be, _ED), lambda i: (i, 0)),
            pl.BlockSpec((_ED, _H), lambda i: (0, 0)),
        ],
        out_specs=[
            pl.BlockSpec((bm, _H), lambda i: (i, 0)),
            pl.BlockSpec((bm, _H), lambda i: (i, 0)),
            pl.BlockSpec((be, _H), lambda i: (i, 0)),
        ],
        out_shape=[
            jax.ShapeDtypeStruct((_N, _H), jnp.float32),
            jax.ShapeDtypeStruct((_N, _H), jnp.float32),
            jax.ShapeDtypeStruct((_E, _H), jnp.float32),
        ],
    )(x, wlT, wrT, edge_attr, weT)


def _combine_bn_proj(parts, dens, b, g, be, wlT, wrT):
    bm = 1000
    nb = _N // bm

    def body(p_ref, d_ref, b_ref, g_ref, be_ref, wl_ref, wr_ref,
             xl_ref, xr_ref, s_ref, h_scr):
        i = pl.program_id(0)

        @pl.when(i == 0)
        def _():
            s_ref[...] = jnp.zeros_like(s_ref)

        @pl.when(i < nb)
        def _():
            raw = p_ref[0] + p_ref[1]
            den = d_ref[0, :, 0] + d_ref[1, :, 0] + 1e-16
            h = raw / den[:, None] + b_ref[...]
            h = jnp.where(h > 0, h, jnp.exp(h) - 1.0)
            h_scr[pl.ds(i * bm, bm), :] = h
            s_ref[0:1, :] += jnp.sum(h, axis=0, keepdims=True)
            s_ref[1:2, :] += jnp.sum(h * h, axis=0, keepdims=True)

        @pl.when(i >= nb)
        def _():
            hb = h_scr[pl.ds((i - nb) * bm, bm), :]
            mu = s_ref[0:1, :] * (1.0 / _N)
            var = s_ref[1:2, :] * (1.0 / _N) - mu * mu
            inv = g_ref[...] * lax.rsqrt(var + 1e-5)
            hb = (hb - mu) * inv + be_ref[...]
            xl_ref[...] = jnp.dot(hb, wl_ref[...],
                                  preferred_element_type=jnp.float32)
            xr_ref[...] = jnp.dot(hb, wr_ref[...],
                                  preferred_element_type=jnp.float32)

    mn9 = lambda i: (0, jnp.minimum(i, nb - 1), 0)
    ph2 = lambda i: (jnp.maximum(i - nb, 0), 0)
    return pl.pallas_call(
        body,
        grid=(2 * nb,),
        in_specs=[
            pl.BlockSpec((_NC, bm, _H), mn9),
            pl.BlockSpec((_NC, bm, _L), mn9),
            pl.BlockSpec((1, _H), lambda i: (0, 0)),
            pl.BlockSpec((1, _H), lambda i: (0, 0)),
            pl.BlockSpec((1, _H), lambda i: (0, 0)),
            pl.BlockSpec((_H, _H), lambda i: (0, 0)),
            pl.BlockSpec((_H, _H), lambda i: (0, 0)),
        ],
        out_specs=[
            pl.BlockSpec((bm, _H), ph2),
            pl.BlockSpec((bm, _H), ph2),
            pl.BlockSpec((2, _H), lambda i: (0, 0)),
        ],
        out_shape=[
            jax.ShapeDtypeStruct((_N, _H), jnp.float32),
            jax.ShapeDtypeStruct((_N, _H), jnp.float32),
            jax.ShapeDtypeStruct((2, _H), jnp.float32),
        ],
        scratch_shapes=[pltpu.VMEM((_N, _H), jnp.float32)],
    )(parts, dens, b, g, be, wlT, wrT)


def _final_combine(parts, dens, b):
    bm = 1000

    def body(p_ref, d_ref, b_ref, o_ref):
        raw = p_ref[0] + p_ref[1]
        den = d_ref[0, :, 0] + d_ref[1, :, 0] + 1e-16
        o_ref[...] = raw / den[:, None] + b_ref[...]

    return pl.pallas_call(
        body,
        grid=(_N // bm,),
        in_specs=[
            pl.BlockSpec((_NC, bm, _H), lambda i: (0, i, 0)),
            pl.BlockSpec((_NC, bm, _L), lambda i: (0, i, 0)),
            pl.BlockSpec((1, _H), lambda i: (0, 0)),
        ],
        out_specs=pl.BlockSpec((bm, _H), lambda i: (i, 0)),
        out_shape=jax.ShapeDtypeStruct((_N, _H), jnp.float32),
    )(parts, dens, b)


def _make_sc_layer(with_el):
    mesh = plsc.VectorSubcoreMesh(
        core_axis_name="c", subcore_axis_name="s",
        num_cores=_NC, num_subcores=_NS)

    scratch = [
        pltpu.VMEM((_SUP, _C), jnp.int32),
        pltpu.VMEM((_SUP, _C), jnp.int32),
        pltpu.VMEM((2, _C, _H), jnp.float32),
        pltpu.VMEM((2, _C, _H), jnp.float32),
        pltpu.VMEM((2, _C, _H), jnp.float32),
        pltpu.VMEM((2, _C, _L), jnp.float32),
        pltpu.VMEM((_H,), jnp.float32),
        pltpu.VMEM_SHARED((_N, _H), jnp.float32),
        pltpu.VMEM_SHARED((_N, _L), jnp.float32),
        pltpu.SemaphoreType.DMA,
        pltpu.SemaphoreType.DMA,
        pltpu.SemaphoreType.DMA,
        pltpu.SemaphoreType.DMA,
    ]
    out_type = (
        jax.ShapeDtypeStruct((_NC, _N, _H), jnp.float32),
        jax.ShapeDtypeStruct((_NC, _N, _L), jnp.float32),
    )

    def body(*refs):
        if with_el:
            (xl_hbm, xr_hbm, el_hbm, src_hbm, dst_hbm, att_hbm,
             zo_hbm, zd_hbm,
             out_hbm, den_hbm,
             srcw_v, dstw_v, xl_v, xr_v, el_v, p_v, att_v,
             out_sh, den_sh, sg0, sg1, ss0, ss1) = refs
        else:
            (xl_hbm, xr_hbm, src_hbm, dst_hbm, att_hbm,
             zo_hbm, zd_hbm,
             out_hbm, den_hbm,
             srcw_v, dstw_v, xl_v, xr_v, el_v, p_v, att_v,
             out_sh, den_sh, sg0, sg1, ss0, ss1) = refs

        cid = lax.axis_index("c")
        sid = lax.axis_index("s")
        wid = sid * _NC + cid
        sg = (sg0, sg1)
        ss = (ss0, ss1)

        pltpu.sync_copy(att_hbm, att_v)
        atts = [att_v[pl.ds(j * _L, _L)] for j in range(_H // _L)]

        @pl.when(sid < _WT)
        def _():
            for i in range(_RPT // _ZR):
                pltpu.sync_copy(zo_hbm, out_sh.at[pl.ds(sid * _RPT + i * _ZR, _ZR)])
                pltpu.sync_copy(zd_hbm, den_sh.at[pl.ds(sid * _RPT + i * _ZR, _ZR)])

        plsc.subcore_barrier()

        def issue_g(s, c, b):
            if with_el:
                off = wid * _EPW + s * (_SUP * _C) + c * _C
                pltpu.async_copy(el_hbm.at[pl.ds(off, _C)], el_v.at[b], sg[b])
            pltpu.async_copy(xl_hbm.at[srcw_v.at[c]], xl_v.at[b], sg[b])
            pltpu.async_copy(xr_hbm.at[dstw_v.at[c]], xr_v.at[b], sg[b])

        def wait_g(b):
            if with_el:
                pltpu.make_async_copy(el_hbm.at[pl.ds(0, _C)], el_v.at[b], sg[b]).wait()
            pltpu.make_async_copy(xl_hbm.at[srcw_v.at[0]], xl_v.at[b], sg[b]).wait()
            pltpu.make_async_copy(xr_hbm.at[dstw_v.at[0]], xr_v.at[b], sg[b]).wait()

        def issue_s(c, b):
            pltpu.async_copy(xr_v.at[b], out_sh.at[dstw_v.at[c]], ss[b], add=True)
            pltpu.async_copy(p_v.at[b], den_sh.at[dstw_v.at[c]], ss[b], add=True)

        def wait_s(b):
            pltpu.make_async_copy(xr_v.at[b], out_sh.at[dstw_v.at[0]], ss[b]).wait()
            pltpu.make_async_copy(p_v.at[b], den_sh.at[dstw_v.at[0]], ss[b]).wait()

        def compute(b):
            xlb, xrb, elb, pb = xl_v.at[b], xr_v.at[b], el_v.at[b], p_v.at[b]

            @plsc.parallel_loop(0, _C, unroll=4)
            def edge_a(i):
                acc = jnp.zeros((_L,), jnp.float32)
                xls = []
                for j in range(_H // _L):
                    sl = pl.ds(j * _L, _L)
                    xlv = xlb[i, sl]
                    xls.append(xlv)
                    mv = xlv + xrb[i, sl]
                    if with_el:
                        mv = mv + elb[i, sl]
                    mv = jnp.maximum(mv, mv * 0.2)
                    acc = acc + mv * atts[j]
                pvec = jnp.exp(jnp.broadcast_to(jnp.sum(acc), (_L,)))
                pb[i, :] = pvec
                for j in range(_H // _L):
                    xrb[i, pl.ds(j * _L, _L)] = xls[j] * pvec

        def superchunk(s, _):
            pltpu.sync_copy(src_hbm.at[wid, s], srcw_v)
            pltpu.sync_copy(dst_hbm.at[wid, s], dstw_v)
            issue_g(s, 0, 0)

            def pair(t, _):
                for b in (0, 1):
                    c = 2 * t + b
                    wait_g(b)

                    @pl.when(c < _SUP - 1)
                    def _():
                        @pl.when(c >= 1)
                        def _():
                            wait_s(1 - b)
                        issue_g(s, c + 1, 1 - b)

                    compute(b)
                    issue_s(c, b)
                return 0
            lax.fori_loop(0, _SUP // 2, pair, 0)
            wait_s(0)
            wait_s(1)
            return 0
        lax.fori_loop(0, _NSUP, superchunk, 0)

        plsc.subcore_barrier()

        @pl.when(sid < _WT)
        def _():
            pltpu.sync_copy(out_sh.at[pl.ds(sid * _RPT, _RPT)],
                            out_hbm.at[cid, pl.ds(sid * _RPT, _RPT)])
            pltpu.sync_copy(den_sh.at[pl.ds(sid * _RPT, _RPT)],
                            den_hbm.at[cid, pl.ds(sid * _RPT, _RPT)])

    return pl.kernel(
        body, out_type=out_type, mesh=mesh, scratch_types=scratch,
        compiler_params=pltpu.CompilerParams(
            needs_layout_passes=False, use_tc_tiling_on_sc=False))


_sc_layer_el = _make_sc_layer(True)
_sc_layer_plain = _make_sc_layer(False)


def kernel(x, edge_index, edge_attr, Wl1, Wr1, We1, att1, b1, g0, be0,
           Wl2, Wr2, att2, b2):
    src = edge_index[0].reshape(_NW, _NSUP, _SUP, _C)
    dst = edge_index[1].reshape(_NW, _NSUP, _SUP, _C)
    zo = jnp.zeros((_ZR, _H), jnp.float32)
    zd = jnp.zeros((_ZR, _L), jnp.float32)

    xl1, xr1, el1 = _proj2_el(x, Wl1.T, Wr1.T, edge_attr, We1.T)
    parts1, dens1 = _sc_layer_el(xl1, xr1, el1, src, dst, att1, zo, zd)
    xl2, xr2, _ = _combine_bn_proj(parts1, dens1, b1.reshape(1, _H),
                                   g0.reshape(1, _H), be0.reshape(1, _H),
                                   Wl2.T, Wr2.T)
    parts2, dens2 = _sc_layer_plain(xl2, xr2, src, dst, att2, zo, zd)
    return _final_combine(parts2, dens2, b2.reshape(1, _H))
